# baseline (speedup 1.0000x reference)
import functools

import jax
import jax.numpy as jnp
from jax import lax
from jax.experimental import pallas as pl
from jax.experimental.pallas import tpu as pltpu
from jax.experimental.pallas import tpu_sc as plsc

D = 64
T = 200
LANES = 16
IDX_MINOR = 128
K_STREAMS = 5
CHUNK = K_STREAMS * IDX_MINOR


def _sc_embed(tok_flat, table, pos_flat):
    n_rows = tok_flat.shape[0]
    info = plsc.get_sparse_core_info()
    nc, ns = info.num_cores, info.num_subcores
    nw = nc * ns
    rows_per_worker = n_rows // nw
    chunks = rows_per_worker // CHUNK
    assert rows_per_worker % CHUNK == 0
    assert rows_per_worker % T == 0

    mesh = plsc.VectorSubcoreMesh(core_axis_name="c", subcore_axis_name="s")

    @functools.partial(
        pl.kernel,
        mesh=mesh,
        compiler_params=pltpu.CompilerParams(use_tc_tiling_on_sc=False),
        out_type=jax.ShapeDtypeStruct((n_rows, D), jnp.float32),
        scratch_types=[
            pltpu.VMEM((CHUNK,), jnp.int32),
            pltpu.VMEM((CHUNK, D), jnp.float32),
            pltpu.VMEM((T * D,), jnp.float32),
            pltpu.SemaphoreType.DMA,
        ],
    )
    def k(tok_hbm, table_hbm, pos_hbm, out_hbm, idx_v, rows_v, pos_v, sem):
        wid = lax.axis_index("s") * nc + lax.axis_index("c")
        pltpu.sync_copy(pos_hbm, pos_v)
        base_row = wid * rows_per_worker

        def chunk_body(cidx, t0):
            pltpu.sync_copy(
                tok_hbm.at[pl.ds(base_row + cidx * CHUNK, CHUNK)],
                idx_v,
            )
            cps = [
                pltpu.async_copy(
                    table_hbm.at[idx_v.at[pl.ds(j * IDX_MINOR, IDX_MINOR)]],
                    rows_v.at[pl.ds(j * IDX_MINOR, IDX_MINOR)],
                    sem,
                )
                for j in range(K_STREAMS)
            ]
            for cp in cps:
                cp.wait()

            def row_body(r, t):
                off = pl.multiple_of(t * D, D)
                for cc in range(D // LANES):
                    cur = rows_v[r, pl.ds(cc * LANES, LANES)]
                    v = pos_v[pl.ds(off + cc * LANES, LANES)]
                    rows_v[r, pl.ds(cc * LANES, LANES)] = cur + v
                return jnp.where(t == T - 1, 0, t + 1)

            t_end = lax.fori_loop(0, CHUNK, row_body, t0, unroll=False)

            pltpu.sync_copy(
                rows_v,
                out_hbm.at[pl.ds(base_row + cidx * CHUNK, CHUNK)],
            )
            return t_end

        lax.fori_loop(0, chunks, chunk_body, jnp.int32(0), unroll=False)

    return k(tok_flat, table, pos_flat)


def kernel(tokens, token_embedding, position_embedding):
    b, t = tokens.shape
    tok_flat = tokens.astype(jnp.int32).reshape(-1)
    pos_flat = position_embedding.reshape(-1)
    out = _sc_embed(tok_flat, token_embedding, pos_flat)
    return out.reshape(b, t, D)


# speedup vs baseline: 1.1635x; 1.1635x over previous
import functools

import jax
import jax.numpy as jnp
from jax import lax
from jax.experimental import pallas as pl
from jax.experimental.pallas import tpu as pltpu
from jax.experimental.pallas import tpu_sc as plsc

D = 64
T = 200
LANES = 16
IDX_MINOR = 128
K_STREAMS = 5
CHUNK = K_STREAMS * IDX_MINOR


def _sc_embed(tok_flat, table, pos_flat):
    n_rows = tok_flat.shape[0]
    info = plsc.get_sparse_core_info()
    nc, ns = info.num_cores, info.num_subcores
    nw = nc * ns
    rows_per_worker = n_rows // nw
    chunks = rows_per_worker // CHUNK
    assert rows_per_worker % CHUNK == 0
    assert rows_per_worker % T == 0
    assert chunks % 2 == 0

    mesh = plsc.VectorSubcoreMesh(core_axis_name="c", subcore_axis_name="s")

    @functools.partial(
        pl.kernel,
        mesh=mesh,
        compiler_params=pltpu.CompilerParams(use_tc_tiling_on_sc=False),
        out_type=jax.ShapeDtypeStruct((n_rows, D), jnp.float32),
        scratch_types=[
            pltpu.VMEM((CHUNK,), jnp.int32),
            pltpu.VMEM((CHUNK,), jnp.int32),
            pltpu.VMEM((CHUNK, D), jnp.float32),
            pltpu.VMEM((CHUNK, D), jnp.float32),
            pltpu.VMEM((T * D,), jnp.float32),
            pltpu.SemaphoreType.DMA,
            pltpu.SemaphoreType.DMA,
            pltpu.SemaphoreType.DMA,
            pltpu.SemaphoreType.DMA,
            pltpu.SemaphoreType.DMA,
            pltpu.SemaphoreType.DMA,
        ],
    )
    def k(tok_hbm, table_hbm, pos_hbm, out_hbm, idx0, idx1, rows0, rows1,
          pos_v, sg0, sg1, si0, si1, so0, so1):
        wid = lax.axis_index("s") * nc + lax.axis_index("c")
        pltpu.sync_copy(pos_hbm, pos_v)
        base = wid * rows_per_worker
        idx_v = (idx0, idx1)
        rows_v = (rows0, rows1)
        sg = (sg0, sg1)
        si = (si0, si1)
        so = (so0, so1)

        def idx_start(c, p):
            pltpu.async_copy(
                tok_hbm.at[pl.ds(base + c * CHUNK, CHUNK)], idx_v[p], si[p])

        def idx_wait(c, p):
            pltpu.make_async_copy(
                tok_hbm.at[pl.ds(base + c * CHUNK, CHUNK)], idx_v[p], si[p]
            ).wait()

        def gathers_start(p):
            for j in range(K_STREAMS):
                pltpu.async_copy(
                    table_hbm.at[idx_v[p].at[pl.ds(j * IDX_MINOR, IDX_MINOR)]],
                    rows_v[p].at[pl.ds(j * IDX_MINOR, IDX_MINOR)],
                    sg[p],
                )

        def gathers_wait(p):
            for j in range(K_STREAMS):
                pltpu.make_async_copy(
                    table_hbm.at[idx_v[p].at[pl.ds(j * IDX_MINOR, IDX_MINOR)]],
                    rows_v[p].at[pl.ds(j * IDX_MINOR, IDX_MINOR)],
                    sg[p],
                ).wait()

        def out_start(c, p):
            pltpu.async_copy(
                rows_v[p], out_hbm.at[pl.ds(base + c * CHUNK, CHUNK)], so[p])

        def out_wait(c, p):
            pltpu.make_async_copy(
                rows_v[p], out_hbm.at[pl.ds(base + c * CHUNK, CHUNK)], so[p]
            ).wait()

        def add_pos(p, t0):
            def row_body(r, t):
                off = pl.multiple_of(t * D, D)
                for cc in range(D // LANES):
                    v = pos_v[pl.ds(off + cc * LANES, LANES)]
                    plsc.addupdate(
                        rows_v[p].at[r, pl.ds(cc * LANES, LANES)], v)
                return jnp.where(t == T - 1, 0, t + 1)

            return lax.fori_loop(0, CHUNK, row_body, t0, unroll=8)

        idx_start(0, 0)
        idx_start(1, 1)
        idx_wait(0, 0)
        gathers_start(0)

        def pair_body(i, t):
            for p in (0, 1):
                g = i * 2 + p

                @pl.when(g + 1 < chunks)
                def _():
                    idx_wait(g + 1, 1 - p)

                    @pl.when(g >= 1)
                    def _():
                        out_wait(g - 1, 1 - p)

                    gathers_start(1 - p)

                gathers_wait(p)
                t = add_pos(p, t)
                out_start(g, p)

                @pl.when(g + 2 < chunks)
                def _():
                    idx_start(g + 2, p)

            return t

        lax.fori_loop(0, chunks // 2, pair_body, jnp.int32(0))
        out_wait(chunks - 2, 0)
        out_wait(chunks - 1, 1)

    return k(tok_flat, table, pos_flat)


def kernel(tokens, token_embedding, position_embedding):
    b, t = tokens.shape
    tok_flat = tokens.astype(jnp.int32).reshape(-1)
    pos_flat = position_embedding.reshape(-1)
    out = _sc_embed(tok_flat, token_embedding, pos_flat)
    return out.reshape(b, t, D)
